# Initial kernel scaffold; baseline (speedup 1.0000x reference)
import jax
import jax.numpy as jnp
from jax.experimental import pallas as pl


def kernel(x, train_x, train_y):
    raise NotImplementedError("write your pallas kernel here")


# speedup vs baseline: 5.4596x; 5.4596x over previous
import functools

import jax
import jax.numpy as jnp
from jax import lax
from jax.experimental import pallas as pl
from jax.experimental.pallas import tpu as pltpu
from jax.experimental.pallas import tpu_sc as plsc

_NC, _NS, _L = 2, 16, 16
_NW = _NC * _NS


_BM = 512


def _argmin_body(x_ref, tt_ref, ind_ref, best_ref, bidx_ref):
    j = pl.program_id(0)
    x = x_ref[...]
    tt = tt_ref[...]
    dots = jnp.dot(x, tt, preferred_element_type=jnp.float32,
                   precision=lax.Precision.HIGHEST)
    tnorm = jnp.sum(tt * tt, axis=0)
    s = tnorm[None, :] - 2.0 * dots
    bmin = jnp.min(s, axis=1)
    iota = lax.broadcasted_iota(jnp.int32, s.shape, 1)
    masked = jnp.where(s == bmin[:, None], iota, jnp.int32(s.shape[1]))
    bidx = jnp.min(masked, axis=1) + j * s.shape[1]

    @pl.when(j == 0)
    def _():
        best_ref[...] = bmin
        bidx_ref[...] = bidx

    @pl.when(j > 0)
    def _():
        prev = best_ref[...]
        take_new = bmin < prev
        best_ref[...] = jnp.where(take_new, bmin, prev)
        bidx_ref[...] = jnp.where(take_new, bidx, bidx_ref[...])

    @pl.when(j == pl.num_programs(0) - 1)
    def _():
        ind_ref[...] = bidx_ref[...]


def _nearest_index(x, train_x_t):
    q = x.shape[0]
    dim, m = train_x_t.shape
    return pl.pallas_call(
        _argmin_body,
        grid=(m // _BM,),
        in_specs=[
            pl.BlockSpec((q, dim), lambda j: (0, 0)),
            pl.BlockSpec((dim, _BM), lambda j: (0, j)),
        ],
        out_specs=pl.BlockSpec((q,), lambda j: (0,)),
        out_shape=jax.ShapeDtypeStruct((q,), jnp.int32),
        scratch_shapes=[
            pltpu.VMEM((q,), jnp.float32),
            pltpu.VMEM((q,), jnp.int32),
        ],
    )(x, train_x_t)


def _label_gather(ind, train_y):
    b = ind.shape[0]
    b_per_w = b // _NW
    mesh = plsc.VectorSubcoreMesh(core_axis_name="c", subcore_axis_name="s")

    @functools.partial(
        pl.kernel,
        mesh=mesh,
        out_type=jax.ShapeDtypeStruct((b,), jnp.float32),
        scratch_types=[
            pltpu.VMEM((b_per_w,), jnp.int32),
            pltpu.VMEM((b_per_w,), jnp.float32),
            pltpu.SemaphoreType.DMA,
        ],
    )
    def gather_kernel(ind_hbm, ty_hbm, out_hbm, idx_v, out_v, sem):
        wid = lax.axis_index("s") * _NC + lax.axis_index("c")
        base = wid * b_per_w
        pltpu.sync_copy(ind_hbm.at[pl.ds(base, b_per_w)], idx_v)
        pltpu.async_copy(ty_hbm.at[idx_v], out_v, sem).wait()
        pltpu.sync_copy(out_v, out_hbm.at[pl.ds(base, b_per_w)])

    return gather_kernel(ind, train_y)


def kernel(x, train_x, train_y):
    ind = _nearest_index(x, train_x.T)
    return _label_gather(ind, train_y)


# speedup vs baseline: 5.8810x; 1.0772x over previous
import functools

import jax
import jax.numpy as jnp
from jax import lax
from jax.experimental import pallas as pl
from jax.experimental.pallas import tpu as pltpu
from jax.experimental.pallas import tpu_sc as plsc

_NC, _NS, _L = 2, 16, 16
_NW = _NC * _NS


_BM = 512


def _argmin_body(x_ref, t_ref, ind_ref, best_ref, bidx_ref):
    j = pl.program_id(0)
    x = x_ref[...]
    tt = t_ref[...].T
    dots = jnp.dot(x, tt, preferred_element_type=jnp.float32,
                   precision=lax.Precision.HIGHEST)
    tnorm = jnp.sum(tt * tt, axis=0)
    s = tnorm[None, :] - 2.0 * dots
    bmin = jnp.min(s, axis=1)
    iota = lax.broadcasted_iota(jnp.int32, s.shape, 1)
    masked = jnp.where(s == bmin[:, None], iota, jnp.int32(s.shape[1]))
    bidx = jnp.min(masked, axis=1) + j * s.shape[1]

    @pl.when(j == 0)
    def _():
        best_ref[...] = bmin
        bidx_ref[...] = bidx

    @pl.when(j > 0)
    def _():
        prev = best_ref[...]
        take_new = bmin < prev
        best_ref[...] = jnp.where(take_new, bmin, prev)
        bidx_ref[...] = jnp.where(take_new, bidx, bidx_ref[...])

    @pl.when(j == pl.num_programs(0) - 1)
    def _():
        ind_ref[...] = bidx_ref[...]


def _nearest_index(x, train_x):
    q = x.shape[0]
    m, dim = train_x.shape
    return pl.pallas_call(
        _argmin_body,
        grid=(m // _BM,),
        in_specs=[
            pl.BlockSpec((q, dim), lambda j: (0, 0)),
            pl.BlockSpec((_BM, dim), lambda j: (j, 0)),
        ],
        out_specs=pl.BlockSpec((q,), lambda j: (0,)),
        out_shape=jax.ShapeDtypeStruct((q,), jnp.int32),
        scratch_shapes=[
            pltpu.VMEM((q,), jnp.float32),
            pltpu.VMEM((q,), jnp.int32),
        ],
    )(x, train_x)


def _label_gather(ind, train_y):
    b = ind.shape[0]
    b_per_w = b // _NW
    mesh = plsc.VectorSubcoreMesh(core_axis_name="c", subcore_axis_name="s")

    @functools.partial(
        pl.kernel,
        mesh=mesh,
        out_type=jax.ShapeDtypeStruct((b,), jnp.float32),
        scratch_types=[
            pltpu.VMEM((b_per_w,), jnp.int32),
            pltpu.VMEM((b_per_w,), jnp.float32),
            pltpu.SemaphoreType.DMA,
        ],
    )
    def gather_kernel(ind_hbm, ty_hbm, out_hbm, idx_v, out_v, sem):
        wid = lax.axis_index("s") * _NC + lax.axis_index("c")
        base = wid * b_per_w
        pltpu.sync_copy(ind_hbm.at[pl.ds(base, b_per_w)], idx_v)
        pltpu.async_copy(ty_hbm.at[idx_v], out_v, sem).wait()
        pltpu.sync_copy(out_v, out_hbm.at[pl.ds(base, b_per_w)])

    return gather_kernel(ind, train_y)


def kernel(x, train_x, train_y):
    ind = _nearest_index(x, train_x)
    return _label_gather(ind, train_y)


# speedup vs baseline: 6.6049x; 1.1231x over previous
import functools

import jax
import jax.numpy as jnp
from jax import lax
from jax.experimental import pallas as pl
from jax.experimental.pallas import tpu as pltpu
from jax.experimental.pallas import tpu_sc as plsc

_NC, _NS, _L = 2, 16, 16
_NW = _NC * _NS


_BM = 2048


def _argmin_body(x_ref, t_ref, ind_ref, best_ref, bidx_ref):
    j = pl.program_id(0)
    x = x_ref[...]
    tt = t_ref[...].T
    dots = jnp.dot(x, tt, preferred_element_type=jnp.float32,
                   precision=lax.Precision.HIGHEST)
    tnorm = jnp.sum(tt * tt, axis=0)
    s = tnorm[None, :] - 2.0 * dots
    bmin = jnp.min(s, axis=1)
    iota = lax.broadcasted_iota(jnp.int32, s.shape, 1)
    masked = jnp.where(s == bmin[:, None], iota, jnp.int32(s.shape[1]))
    bidx = jnp.min(masked, axis=1) + j * s.shape[1]

    @pl.when(j == 0)
    def _():
        best_ref[...] = bmin
        bidx_ref[...] = bidx

    @pl.when(j > 0)
    def _():
        prev = best_ref[...]
        take_new = bmin < prev
        best_ref[...] = jnp.where(take_new, bmin, prev)
        bidx_ref[...] = jnp.where(take_new, bidx, bidx_ref[...])

    @pl.when(j == pl.num_programs(0) - 1)
    def _():
        ind_ref[...] = bidx_ref[...]


def _nearest_index(x, train_x):
    q = x.shape[0]
    m, dim = train_x.shape
    return pl.pallas_call(
        _argmin_body,
        grid=(m // _BM,),
        in_specs=[
            pl.BlockSpec((q, dim), lambda j: (0, 0)),
            pl.BlockSpec((_BM, dim), lambda j: (j, 0)),
        ],
        out_specs=pl.BlockSpec((q,), lambda j: (0,)),
        out_shape=jax.ShapeDtypeStruct((q,), jnp.int32),
        scratch_shapes=[
            pltpu.VMEM((q,), jnp.float32),
            pltpu.VMEM((q,), jnp.int32),
        ],
    )(x, train_x)


def _label_gather(ind, train_y):
    b = ind.shape[0]
    b_per_w = b // _NW
    mesh = plsc.VectorSubcoreMesh(core_axis_name="c", subcore_axis_name="s")

    @functools.partial(
        pl.kernel,
        mesh=mesh,
        out_type=jax.ShapeDtypeStruct((b,), jnp.float32),
        scratch_types=[
            pltpu.VMEM((b_per_w,), jnp.int32),
            pltpu.VMEM((b_per_w,), jnp.float32),
            pltpu.SemaphoreType.DMA,
        ],
    )
    def gather_kernel(ind_hbm, ty_hbm, out_hbm, idx_v, out_v, sem):
        wid = lax.axis_index("s") * _NC + lax.axis_index("c")
        base = wid * b_per_w
        pltpu.sync_copy(ind_hbm.at[pl.ds(base, b_per_w)], idx_v)
        pltpu.async_copy(ty_hbm.at[idx_v], out_v, sem).wait()
        pltpu.sync_copy(out_v, out_hbm.at[pl.ds(base, b_per_w)])

    return gather_kernel(ind, train_y)


def kernel(x, train_x, train_y):
    ind = _nearest_index(x, train_x)
    return _label_gather(ind, train_y)


# speedup vs baseline: 6.7779x; 1.0262x over previous
import functools

import jax
import jax.numpy as jnp
from jax import lax
from jax.experimental import pallas as pl
from jax.experimental.pallas import tpu as pltpu
from jax.experimental.pallas import tpu_sc as plsc

_NC, _NS, _L = 2, 16, 16
_NW = _NC * _NS


_BM = 2048


def _argmin_body(x_ref, t_ref, ind_ref):
    t = t_ref[...]
    xt = x_ref[...].T
    dots = jnp.dot(t, xt, preferred_element_type=jnp.float32,
                   precision=lax.Precision.HIGHEST)
    tnorm = jnp.sum(t * t, axis=1)
    s = tnorm[:, None] - 2.0 * dots
    bmin = jnp.min(s, axis=0)
    iota = lax.broadcasted_iota(jnp.int32, s.shape, 0)
    masked = jnp.where(s == bmin[None, :], iota, jnp.int32(s.shape[0]))
    ind_ref[...] = jnp.min(masked, axis=0)


def _nearest_index(x, train_x):
    q = x.shape[0]
    m, dim = train_x.shape
    return pl.pallas_call(
        _argmin_body,
        out_shape=jax.ShapeDtypeStruct((q,), jnp.int32),
    )(x, train_x)


def _label_gather(ind, train_y):
    b = ind.shape[0]
    b_per_w = b // _NW
    mesh = plsc.VectorSubcoreMesh(core_axis_name="c", subcore_axis_name="s")

    @functools.partial(
        pl.kernel,
        mesh=mesh,
        out_type=jax.ShapeDtypeStruct((b,), jnp.float32),
        scratch_types=[
            pltpu.VMEM((b_per_w,), jnp.int32),
            pltpu.VMEM((b_per_w,), jnp.float32),
            pltpu.SemaphoreType.DMA,
        ],
    )
    def gather_kernel(ind_hbm, ty_hbm, out_hbm, idx_v, out_v, sem):
        wid = lax.axis_index("s") * _NC + lax.axis_index("c")
        base = wid * b_per_w
        pltpu.sync_copy(ind_hbm.at[pl.ds(base, b_per_w)], idx_v)
        pltpu.async_copy(ty_hbm.at[idx_v], out_v, sem).wait()
        pltpu.sync_copy(out_v, out_hbm.at[pl.ds(base, b_per_w)])

    return gather_kernel(ind, train_y)


def kernel(x, train_x, train_y):
    ind = _nearest_index(x, train_x)
    return _label_gather(ind, train_y)


# speedup vs baseline: 7.1764x; 1.0588x over previous
import functools

import jax
import jax.numpy as jnp
from jax import lax
from jax.experimental import pallas as pl
from jax.experimental.pallas import tpu as pltpu
from jax.experimental.pallas import tpu_sc as plsc

_NC, _NS, _L = 1, 16, 16
_NW = _NC * _NS


_BM = 2048


def _argmin_body(x_ref, t_ref, ind_ref):
    t = t_ref[...]
    xt = x_ref[...].T
    dots = jnp.dot(t, xt, preferred_element_type=jnp.float32,
                   precision=lax.Precision.HIGHEST)
    tnorm = jnp.sum(t * t, axis=1)
    s = tnorm[:, None] - 2.0 * dots
    bmin = jnp.min(s, axis=0)
    iota = lax.broadcasted_iota(jnp.int32, s.shape, 0)
    masked = jnp.where(s == bmin[None, :], iota, jnp.int32(s.shape[0]))
    ind_ref[...] = jnp.min(masked, axis=0)


def _nearest_index(x, train_x):
    q = x.shape[0]
    m, dim = train_x.shape
    return pl.pallas_call(
        _argmin_body,
        out_shape=jax.ShapeDtypeStruct((q,), jnp.int32),
    )(x, train_x)


def _label_gather(ind, train_y):
    b = ind.shape[0]
    b_per_w = b // _NW
    mesh = plsc.VectorSubcoreMesh(core_axis_name="c", subcore_axis_name="s", num_cores=1)

    @functools.partial(
        pl.kernel,
        mesh=mesh,
        out_type=jax.ShapeDtypeStruct((b,), jnp.float32),
        scratch_types=[
            pltpu.VMEM((b_per_w,), jnp.int32),
            pltpu.VMEM((b_per_w,), jnp.float32),
            pltpu.SemaphoreType.DMA,
        ],
    )
    def gather_kernel(ind_hbm, ty_hbm, out_hbm, idx_v, out_v, sem):
        wid = lax.axis_index("s") * _NC + lax.axis_index("c")
        base = wid * b_per_w
        pltpu.sync_copy(ind_hbm.at[pl.ds(base, b_per_w)], idx_v)
        pltpu.async_copy(ty_hbm.at[idx_v], out_v, sem).wait()
        pltpu.sync_copy(out_v, out_hbm.at[pl.ds(base, b_per_w)])

    return gather_kernel(ind, train_y)


def kernel(x, train_x, train_y):
    ind = _nearest_index(x, train_x)
    return _label_gather(ind, train_y)
